# baseline (speedup 1.0000x reference)
import functools

import jax
import jax.numpy as jnp
from jax import lax
from jax.experimental import pallas as pl
from jax.experimental.pallas import tpu as pltpu
from jax.experimental.pallas import tpu_sc as plsc

N = 10000
E = 320000
D = 128

NC = 2
NS = 16
NT = NC * NS
CHUNK = 128
F_CHUNKS = 160
S_CHUNKS = 0
TCHUNKS = NS * (F_CHUNKS + S_CHUNKS)
EPAD = TCHUNKS * CHUNK
NA = N + 80
RB = 80
NZB = NA // RB
NWB = N // RB


IGRP = 32


def _agg_body(x_hbm, src_hbm, dst_hbm, zeros_hbm, out_hbm,
              src_v, dst_v, r0, r1, agg_sh, g0, g1, s0, s1):
    c = lax.axis_index("c")
    s = lax.axis_index("s")
    rows = (r0, r1)
    gsem = (g0, g1)
    ssem = (s0, s1)
    nst = jnp.where(c == 0, F_CHUNKS // IGRP, S_CHUNKS // IGRP)
    base0 = jnp.where(c == 0, s * F_CHUNKS, NS * F_CHUNKS + s * S_CHUNKS)

    zbuf = r0.at[pl.ds(0, RB)]
    pltpu.sync_copy(zeros_hbm, zbuf)
    for k in range(8):
        j = s + NS * k

        @pl.when(j < NZB)
        def _():
            pltpu.sync_copy(zbuf, agg_sh.at[pl.ds(j * RB, RB)])

    plsc.subcore_barrier()

    def gather(i, b):
        pltpu.async_copy(x_hbm.at[src_v.at[i]], rows[b], gsem[b])

    def gather_wait(i, b):
        pltpu.make_async_copy(x_hbm.at[src_v.at[i]], rows[b], gsem[b]).wait()

    def scat(i, b):
        pltpu.async_copy(rows[b], agg_sh.at[dst_v.at[i]], ssem[b], add=True)

    def scat_wait(i, b):
        pltpu.make_async_copy(rows[b], agg_sh.at[dst_v.at[i]], ssem[b]).wait()

    def stage(st, carry):
        cb = base0 + st * IGRP
        pltpu.sync_copy(src_hbm.at[pl.ds(cb, IGRP)], src_v)
        pltpu.sync_copy(dst_hbm.at[pl.ds(cb, IGRP)], dst_v)

        gather(0, 0)
        gather_wait(0, 0)
        scat(0, 0)
        gather(1, 1)

        def pair(g, carry):
            i = 2 * g + 1
            gather_wait(i, 1)
            scat(i, 1)
            scat_wait(i - 1, 0)
            gather(i + 1, 0)
            gather_wait(i + 1, 0)
            scat(i + 1, 0)
            scat_wait(i, 1)
            gather(i + 2, 1)
            return carry

        lax.fori_loop(0, (IGRP - 2) // 2, pair, 0)

        gather_wait(IGRP - 1, 1)
        scat(IGRP - 1, 1)
        scat_wait(IGRP - 2, 0)
        scat_wait(IGRP - 1, 1)
        return carry

    lax.fori_loop(0, nst, stage, 0)
    plsc.subcore_barrier()

    wbuf = r1.at[pl.ds(0, RB)]
    for k in range(8):
        j = s + NS * k

        @pl.when(j < NWB)
        def _():
            pltpu.sync_copy(agg_sh.at[pl.ds(j * RB, RB)], wbuf)
            pltpu.sync_copy(wbuf, out_hbm.at[c, pl.ds(j * RB, RB)])


_sc_agg = functools.partial(
    pl.kernel,
    out_type=jax.ShapeDtypeStruct((NC, N, D), jnp.float32),
    mesh=plsc.VectorSubcoreMesh(core_axis_name="c", subcore_axis_name="s"),
    scratch_types=[
        pltpu.VMEM((IGRP, CHUNK), jnp.int32),
        pltpu.VMEM((IGRP, CHUNK), jnp.int32),
        pltpu.VMEM((CHUNK, D), jnp.float32),
        pltpu.VMEM((CHUNK, D), jnp.float32),
        pltpu.VMEM_SHARED((NA, D), jnp.float32),
    ] + [pltpu.SemaphoreType.DMA] * 4,
)(_agg_body)


def _mlp_body(x_ref, a0_ref, a1_ref, w1_ref, b1_ref, w2_ref, b2_ref, o_ref):
    h = x_ref[...] + a0_ref[...] + a1_ref[...]
    h = jnp.dot(h, w1_ref[...], preferred_element_type=jnp.float32) + b1_ref[...]
    h = jnp.maximum(h, 0.0)
    h = jnp.dot(h, w2_ref[...], preferred_element_type=jnp.float32) + b2_ref[...]
    o_ref[...] = jnp.maximum(h, 0.0)


_BLK = 1000


def _mlp(x, a0, a1, w1, b1, w2, b2):
    row_spec = pl.BlockSpec((_BLK, D), lambda i: (i, 0))
    full_spec = pl.BlockSpec((D, D), lambda i: (0, 0))
    bias_spec = pl.BlockSpec((1, D), lambda i: (0, 0))
    return pl.pallas_call(
        _mlp_body,
        grid=(N // _BLK,),
        in_specs=[row_spec, row_spec, row_spec,
                  full_spec, bias_spec, full_spec, bias_spec],
        out_specs=row_spec,
        out_shape=jax.ShapeDtypeStruct((N, D), jnp.float32),
    )(x, a0, a1, w1, b1, w2, b2)


def kernel(features, edge_index, W1, b1, W2, b2):
    pad = EPAD - E
    src = jnp.concatenate(
        [edge_index[0], jnp.zeros((pad,), jnp.int32)]
    ).reshape(TCHUNKS, CHUNK)
    dst = jnp.concatenate(
        [edge_index[1], jnp.full((pad,), N, jnp.int32)]
    ).reshape(TCHUNKS, CHUNK)
    zeros = jnp.zeros((RB, D), jnp.float32)
    x = features
    L = W1.shape[0]
    for l in range(L):
        parts = _sc_agg(x, src, dst, zeros)
        x = _mlp(x, parts[0], parts[1], W1[l], b1[l].reshape(1, D),
                 W2[l], b2[l].reshape(1, D))
    return x


# speedup vs baseline: 2.0026x; 2.0026x over previous
import functools

import jax
import jax.numpy as jnp
from jax import lax
from jax.experimental import pallas as pl
from jax.experimental.pallas import tpu as pltpu
from jax.experimental.pallas import tpu_sc as plsc

N = 10000
E = 320000
D = 128

NC = 2
NS = 16
NT = NC * NS
CHUNK = 128
F_CHUNKS = 112
S_CHUNKS = 48
TCHUNKS = NS * (F_CHUNKS + S_CHUNKS)
EPAD = TCHUNKS * CHUNK
NA = N + 80
RB = 80
NZB = NA // RB
NWB = N // RB


IGRP = 32


def _agg_body(x_hbm, src_hbm, dst_hbm, zeros_hbm, out_hbm,
              src_v, dst_v, r0, r1, agg_sh, g0, g1, s0, s1):
    c = lax.axis_index("c")
    s = lax.axis_index("s")
    rows = (r0, r1)
    gsem = (g0, g1)
    ssem = (s0, s1)
    nst = jnp.where(c == 0, F_CHUNKS // IGRP, S_CHUNKS // IGRP)
    base0 = jnp.where(c == 0, s * F_CHUNKS, NS * F_CHUNKS + s * S_CHUNKS)

    zbuf = r0.at[pl.ds(0, RB)]
    pltpu.sync_copy(zeros_hbm, zbuf)
    for k in range(8):
        j = s + NS * k

        @pl.when(j < NZB)
        def _():
            pltpu.sync_copy(zbuf, agg_sh.at[pl.ds(j * RB, RB)])

    plsc.subcore_barrier()

    def gather(i, b):
        pltpu.async_copy(x_hbm.at[src_v.at[i]], rows[b], gsem[b])

    def gather_wait(i, b):
        pltpu.make_async_copy(x_hbm.at[src_v.at[i]], rows[b], gsem[b]).wait()

    def scat(i, b):
        pltpu.async_copy(rows[b], agg_sh.at[dst_v.at[i]], ssem[b], add=True)

    def scat_wait(i, b):
        pltpu.make_async_copy(rows[b], agg_sh.at[dst_v.at[i]], ssem[b]).wait()

    def stage(st, carry):
        cb = base0 + st * IGRP
        pltpu.sync_copy(src_hbm.at[pl.ds(cb, IGRP)], src_v)
        pltpu.sync_copy(dst_hbm.at[pl.ds(cb, IGRP)], dst_v)

        gather(0, 0)
        gather_wait(0, 0)
        scat(0, 0)
        gather(1, 1)

        def pair(g, carry):
            i = 2 * g + 1
            gather_wait(i, 1)
            scat(i, 1)
            scat_wait(i - 1, 0)
            gather(i + 1, 0)
            gather_wait(i + 1, 0)
            scat(i + 1, 0)
            scat_wait(i, 1)
            gather(i + 2, 1)
            return carry

        lax.fori_loop(0, (IGRP - 2) // 2, pair, 0)

        gather_wait(IGRP - 1, 1)
        scat(IGRP - 1, 1)
        scat_wait(IGRP - 2, 0)
        scat_wait(IGRP - 1, 1)
        return carry

    lax.fori_loop(0, nst, stage, 0)
    plsc.subcore_barrier()

    wbuf = r1.at[pl.ds(0, RB)]
    for k in range(8):
        j = s + NS * k

        @pl.when(j < NWB)
        def _():
            pltpu.sync_copy(agg_sh.at[pl.ds(j * RB, RB)], wbuf)
            pltpu.sync_copy(wbuf, out_hbm.at[c, pl.ds(j * RB, RB)])


_sc_agg = functools.partial(
    pl.kernel,
    out_type=jax.ShapeDtypeStruct((NC, N, D), jnp.float32),
    mesh=plsc.VectorSubcoreMesh(core_axis_name="c", subcore_axis_name="s"),
    scratch_types=[
        pltpu.VMEM((IGRP, CHUNK), jnp.int32),
        pltpu.VMEM((IGRP, CHUNK), jnp.int32),
        pltpu.VMEM((CHUNK, D), jnp.float32),
        pltpu.VMEM((CHUNK, D), jnp.float32),
        pltpu.VMEM_SHARED((NA, D), jnp.float32),
    ] + [pltpu.SemaphoreType.DMA] * 4,
)(_agg_body)


def _mlp_body(x_ref, a0_ref, a1_ref, w1_ref, b1_ref, w2_ref, b2_ref, o_ref):
    h = x_ref[...] + a0_ref[...] + a1_ref[...]
    h = jnp.dot(h, w1_ref[...], preferred_element_type=jnp.float32) + b1_ref[...]
    h = jnp.maximum(h, 0.0)
    h = jnp.dot(h, w2_ref[...], preferred_element_type=jnp.float32) + b2_ref[...]
    o_ref[...] = jnp.maximum(h, 0.0)


_BLK = 1000


def _mlp(x, a0, a1, w1, b1, w2, b2):
    row_spec = pl.BlockSpec((_BLK, D), lambda i: (i, 0))
    full_spec = pl.BlockSpec((D, D), lambda i: (0, 0))
    bias_spec = pl.BlockSpec((1, D), lambda i: (0, 0))
    return pl.pallas_call(
        _mlp_body,
        grid=(N // _BLK,),
        in_specs=[row_spec, row_spec, row_spec,
                  full_spec, bias_spec, full_spec, bias_spec],
        out_specs=row_spec,
        out_shape=jax.ShapeDtypeStruct((N, D), jnp.float32),
    )(x, a0, a1, w1, b1, w2, b2)


def kernel(features, edge_index, W1, b1, W2, b2):
    pad = EPAD - E
    src = jnp.concatenate(
        [edge_index[0], jnp.zeros((pad,), jnp.int32)]
    ).reshape(TCHUNKS, CHUNK)
    dst = jnp.concatenate(
        [edge_index[1], jnp.full((pad,), N, jnp.int32)]
    ).reshape(TCHUNKS, CHUNK)
    zeros = jnp.zeros((RB, D), jnp.float32)
    x = features
    L = W1.shape[0]
    for l in range(L):
        parts = _sc_agg(x, src, dst, zeros)
        x = _mlp(x, parts[0], parts[1], W1[l], b1[l].reshape(1, D),
                 W2[l], b2[l].reshape(1, D))
    return x
